# Initial kernel scaffold; baseline (speedup 1.0000x reference)
import functools

import jax
import jax.numpy as jnp
from jax import lax
from jax.experimental import pallas as pl
from jax.experimental.pallas import tpu as pltpu
from jax.experimental.pallas import tpu_sc as plsc

EMBED_DIM = 64
NUM_WORKERS = 32
IDX_PER_XFER = 128
XFERS_PER_CHUNK = 8
CHUNK = IDX_PER_XFER * XFERS_PER_CHUNK


def _gather_body(idx_hbm, table_hbm, out_hbm, idx_v, rows_v, sem, *,
                 chunks_per_worker):
  wid = lax.axis_index("s") * 2 + lax.axis_index("c")
  row0 = wid * (chunks_per_worker * XFERS_PER_CHUNK)

  def body(g, carry):
    r = pl.multiple_of(row0 + g * XFERS_PER_CHUNK, XFERS_PER_CHUNK)
    pltpu.sync_copy(idx_hbm.at[pl.ds(r, XFERS_PER_CHUNK)], idx_v)
    copies = []
    for j in range(XFERS_PER_CHUNK):
      copies.append(
          pltpu.async_copy(
              table_hbm.at[idx_v.at[j]],
              rows_v.at[pl.ds(j * IDX_PER_XFER, IDX_PER_XFER)],
              sem,
          ))
    for c in copies:
      c.wait()
    out_base = pl.multiple_of(r * IDX_PER_XFER, CHUNK)
    pltpu.sync_copy(rows_v, out_hbm.at[pl.ds(out_base, CHUNK)])
    return carry

  lax.fori_loop(0, chunks_per_worker, body, 0, unroll=False)


def kernel(indices, table):
  batch, hist = indices.shape
  total = batch * hist
  assert total % (NUM_WORKERS * CHUNK) == 0
  chunks_per_worker = total // (NUM_WORKERS * CHUNK)

  idx2d = indices.astype(jnp.int32).reshape(total // IDX_PER_XFER,
                                            IDX_PER_XFER)

  mesh = plsc.VectorSubcoreMesh(core_axis_name="c", subcore_axis_name="s")
  gather = functools.partial(
      pl.kernel,
      mesh=mesh,
      out_type=jax.ShapeDtypeStruct((total, EMBED_DIM), jnp.float32),
      scratch_types=[
          pltpu.VMEM((XFERS_PER_CHUNK, IDX_PER_XFER), jnp.int32),
          pltpu.VMEM((CHUNK, EMBED_DIM), jnp.float32),
          pltpu.SemaphoreType.DMA,
      ],
  )(functools.partial(_gather_body, chunks_per_worker=chunks_per_worker))

  out = gather(idx2d, table)
  return out.reshape(batch, hist, EMBED_DIM)


# speedup vs baseline: 1.8444x; 1.8444x over previous
import functools

import jax
import jax.numpy as jnp
from jax import lax
from jax.experimental import pallas as pl
from jax.experimental.pallas import tpu as pltpu
from jax.experimental.pallas import tpu_sc as plsc

EMBED_DIM = 64
NUM_WORKERS = 32
IDX_PER_XFER = 128
XFERS_PER_CHUNK = 8
CHUNK = IDX_PER_XFER * XFERS_PER_CHUNK


def _gather_body(idx_hbm, table_hbm, out_hbm, idx_v, rows_v, sem, *,
                 chunks_per_worker):
  wid = lax.axis_index("s") * 2 + lax.axis_index("c")
  row0 = wid * (chunks_per_worker * XFERS_PER_CHUNK)

  def body(g, carry):
    r = pl.multiple_of(row0 + g * XFERS_PER_CHUNK, XFERS_PER_CHUNK)
    pltpu.sync_copy(idx_hbm.at[pl.ds(r, XFERS_PER_CHUNK)], idx_v)
    copies = []
    for j in range(XFERS_PER_CHUNK):
      copies.append(
          pltpu.async_copy(
              table_hbm.at[idx_v.at[j]],
              rows_v.at[pl.ds(j * IDX_PER_XFER, IDX_PER_XFER)],
              sem,
          ))
    for c in copies:
      c.wait()
    out_base = pl.multiple_of(r * IDX_PER_XFER, CHUNK)
    pltpu.sync_copy(rows_v, out_hbm.at[pl.ds(out_base, CHUNK)])
    return carry

  lax.fori_loop(0, chunks_per_worker, body, 0, unroll=False)


def kernel(indices, table):
  batch, hist = indices.shape
  total = batch * hist
  assert total % (NUM_WORKERS * CHUNK) == 0
  chunks_per_worker = total // (NUM_WORKERS * CHUNK)

  idx2d = indices.astype(jnp.int32).reshape(total // IDX_PER_XFER,
                                            IDX_PER_XFER)

  mesh = plsc.VectorSubcoreMesh(core_axis_name="c", subcore_axis_name="s")
  gather = functools.partial(
      pl.kernel,
      mesh=mesh,
      out_type=jax.ShapeDtypeStruct((total, EMBED_DIM), jnp.float32),
      scratch_types=[
          pltpu.VMEM((XFERS_PER_CHUNK, IDX_PER_XFER), jnp.int32),
          pltpu.VMEM((CHUNK, EMBED_DIM), jnp.float32),
          pltpu.SemaphoreType.DMA,
      ],
      compiler_params=pltpu.CompilerParams(use_tc_tiling_on_sc=False),
  )(functools.partial(_gather_body, chunks_per_worker=chunks_per_worker))

  out = gather(idx2d, table)
  return out.reshape(batch, hist, EMBED_DIM)


# speedup vs baseline: 1.8711x; 1.0144x over previous
import functools

import jax
import jax.numpy as jnp
from jax import lax
from jax.experimental import pallas as pl
from jax.experimental.pallas import tpu as pltpu
from jax.experimental.pallas import tpu_sc as plsc

EMBED_DIM = 64
NUM_WORKERS = 32
IDX_PER_XFER = 128
XFERS_PER_CHUNK = 4
CHUNK = IDX_PER_XFER * XFERS_PER_CHUNK
NBUF = 2


def _gather_body(idx_hbm, table_hbm, out_hbm, idx_v, rows_v, sem_idx,
                 sem_gather, sem_out, *, chunks_per_worker):
  wid = lax.axis_index("s") * 2 + lax.axis_index("c")
  row0 = wid * (chunks_per_worker * XFERS_PER_CHUNK)
  n = chunks_per_worker

  def idx_rows(g):
    return pl.multiple_of(row0 + g * XFERS_PER_CHUNK, XFERS_PER_CHUNK)

  def start_idx_load(g, b):
    pltpu.async_copy(idx_hbm.at[pl.ds(idx_rows(g), XFERS_PER_CHUNK)],
                     idx_v.at[b], sem_idx.at[b])

  def wait_idx_load(g, b):
    pltpu.make_async_copy(idx_hbm.at[pl.ds(idx_rows(g), XFERS_PER_CHUNK)],
                          idx_v.at[b], sem_idx.at[b]).wait()

  def gather_descs(b):
    return [
        pltpu.make_async_copy(
            table_hbm.at[idx_v.at[b, j]],
            rows_v.at[b, pl.ds(j * IDX_PER_XFER, IDX_PER_XFER)],
            sem_gather.at[b],
        ) for j in range(XFERS_PER_CHUNK)
    ]

  def out_desc(g, b):
    out_base = pl.multiple_of(idx_rows(g) * IDX_PER_XFER, CHUNK)
    return pltpu.make_async_copy(rows_v.at[b],
                                 out_hbm.at[pl.ds(out_base, CHUNK)],
                                 sem_out.at[b])

  start_idx_load(0, 0)

  def body(g, carry):
    b = lax.rem(g, NBUF)
    @pl.when(g >= NBUF)
    def _():
      out_desc(g - NBUF, b).wait()

    wait_idx_load(g, b)
    for d in gather_descs(b):
      d.start()

    @pl.when(g + 1 < n)
    def _():
      start_idx_load(g + 1, 1 - b)

    for d in gather_descs(b):
      d.wait()
    out_desc(g, b).start()
    return carry

  lax.fori_loop(0, n, body, 0, unroll=False)

  for k in range(NBUF):
    g = n - NBUF + k
    out_desc(g, lax.rem(g, NBUF)).wait()


def kernel(indices, table):
  batch, hist = indices.shape
  total = batch * hist
  assert total % (NUM_WORKERS * CHUNK) == 0
  chunks_per_worker = total // (NUM_WORKERS * CHUNK)

  idx3d = indices.astype(jnp.int32).reshape(
      total // (XFERS_PER_CHUNK * IDX_PER_XFER), XFERS_PER_CHUNK,
      IDX_PER_XFER)
  idx2d = idx3d.reshape(total // IDX_PER_XFER, IDX_PER_XFER)

  mesh = plsc.VectorSubcoreMesh(core_axis_name="c", subcore_axis_name="s")
  gather = functools.partial(
      pl.kernel,
      mesh=mesh,
      out_type=jax.ShapeDtypeStruct((total, EMBED_DIM), jnp.float32),
      scratch_types=[
          pltpu.VMEM((NBUF, XFERS_PER_CHUNK, IDX_PER_XFER), jnp.int32),
          pltpu.VMEM((NBUF, CHUNK, EMBED_DIM), jnp.float32),
          pltpu.SemaphoreType.DMA((NBUF,)),
          pltpu.SemaphoreType.DMA((NBUF,)),
          pltpu.SemaphoreType.DMA((NBUF,)),
      ],
      compiler_params=pltpu.CompilerParams(use_tc_tiling_on_sc=False),
  )(functools.partial(_gather_body, chunks_per_worker=chunks_per_worker))

  out = gather(idx2d, table)
  return out.reshape(batch, hist, EMBED_DIM)
